# Initial kernel scaffold; baseline (speedup 1.0000x reference)
import functools

import jax
import jax.numpy as jnp
from jax import lax
from jax.experimental import pallas as pl
from jax.experimental.pallas import tpu as pltpu
from jax.experimental.pallas import tpu_sc as plsc

N = 10000
E = 320000
D = 128

NC = 2
NS = 16
NW = NC * NS

N_PAD = 10240
RPT = N_PAD // NS
EPT = E // NW
K = 400
SUB = 80
NSUB = K // SUB
NCHUNK = EPT // K
DEG_W = 16

_mesh = plsc.VectorSubcoreMesh(core_axis_name="c", subcore_axis_name="s")


def _sc_agg_body(with_deg, *refs):
    if with_deg:
        (x_hbm, edge_hbm, agg_out, deg_out,
         rows, srcb, dstb, onesb, accum, degacc, sem) = refs
    else:
        (x_hbm, edge_hbm, agg_out,
         rows, srcb, dstb, accum, sem) = refs

    c = lax.axis_index("c")
    s = lax.axis_index("s")
    wid = c * NS + s
    rbase = s * RPT

    def zrow(i, carry):
        for j in range(D // 16):
            rows[i, pl.ds(j * 16, 16)] = jnp.zeros((16,), jnp.float32)
        return carry

    lax.fori_loop(0, K, zrow, 0)
    pltpu.sync_copy(rows, accum.at[pl.ds(rbase, K)])
    pltpu.sync_copy(rows.at[pl.ds(0, RPT - K)],
                    accum.at[pl.ds(rbase + K, RPT - K)])

    if with_deg:
        def zdeg(i, carry):
            onesb[i] = jnp.zeros((16,), jnp.float32)
            return carry

        lax.fori_loop(0, K, zdeg, 0)
        pltpu.sync_copy(onesb, degacc.at[pl.ds(rbase, K)])
        pltpu.sync_copy(onesb.at[pl.ds(0, RPT - K)],
                        degacc.at[pl.ds(rbase + K, RPT - K)])

        def fones(i, carry):
            onesb[i] = jnp.ones((16,), jnp.float32)
            return carry

        lax.fori_loop(0, K, fones, 0)

    plsc.subcore_barrier()

    def chunk(i, carry):
        ebase = wid * EPT + i * K
        pltpu.sync_copy(edge_hbm.at[0, pl.ds(ebase, K)], srcb)
        for j in range(NSUB):
            pltpu.sync_copy(edge_hbm.at[1, pl.ds(ebase + j * SUB, SUB)],
                            dstb.at[j])
        pltpu.async_copy(x_hbm.at[srcb], rows, sem).wait()
        for j in range(NSUB):
            pltpu.sync_copy(rows.at[pl.ds(j * SUB, SUB)],
                            accum.at[dstb.at[j]], add=True)
        if with_deg:
            for j in range(NSUB):
                pltpu.sync_copy(onesb.at[pl.ds(0, SUB)],
                                degacc.at[dstb.at[j]], add=True)
        return carry

    lax.fori_loop(0, NCHUNK, chunk, 0)

    plsc.subcore_barrier()

    pltpu.sync_copy(accum.at[pl.ds(rbase, RPT)],
                    agg_out.at[c, pl.ds(rbase, RPT)])
    if with_deg:
        pltpu.sync_copy(degacc.at[pl.ds(rbase, RPT)],
                        deg_out.at[c, pl.ds(rbase, RPT)])


_sc_agg_deg = pl.kernel(
    functools.partial(_sc_agg_body, True),
    out_type=[
        jax.ShapeDtypeStruct((NC, N_PAD, D), jnp.float32),
        jax.ShapeDtypeStruct((NC, N_PAD, DEG_W), jnp.float32),
    ],
    mesh=_mesh,
    scratch_types=[
        pltpu.VMEM((K, D), jnp.float32),
        pltpu.VMEM((K,), jnp.int32),
        pltpu.VMEM((NSUB, SUB), jnp.int32),
        pltpu.VMEM((K, DEG_W), jnp.float32),
        pltpu.VMEM_SHARED((N_PAD, D), jnp.float32),
        pltpu.VMEM_SHARED((N_PAD, DEG_W), jnp.float32),
        pltpu.SemaphoreType.DMA,
    ],
)

_sc_agg = pl.kernel(
    functools.partial(_sc_agg_body, False),
    out_type=[jax.ShapeDtypeStruct((NC, N_PAD, D), jnp.float32)],
    mesh=_mesh,
    scratch_types=[
        pltpu.VMEM((K, D), jnp.float32),
        pltpu.VMEM((K,), jnp.int32),
        pltpu.VMEM((NSUB, SUB), jnp.int32),
        pltpu.VMEM_SHARED((N_PAD, D), jnp.float32),
        pltpu.SemaphoreType.DMA,
    ],
)

R = 1000


def _tc_layer_body(act, h_ref, p_ref, d_ref, w_ref, b_ref, o_ref):
    p = p_ref[0] + p_ref[1]
    deg = d_ref[0, :, 0:1] + d_ref[1, :, 0:1]
    agg = p / jnp.maximum(deg, 1.0)
    z = jnp.concatenate([h_ref[...], agg], axis=1)
    out = jnp.dot(z, w_ref[...], preferred_element_type=jnp.float32)
    out = out + b_ref[...]
    if act:
        out = jnp.maximum(out, 0.0)
    o_ref[...] = out


def _tc_layer(act, h, agg_p, deg_p, w, b):
    return pl.pallas_call(
        functools.partial(_tc_layer_body, act),
        grid=(N // R,),
        in_specs=[
            pl.BlockSpec((R, D), lambda i: (i, 0)),
            pl.BlockSpec((NC, R, D), lambda i: (0, i, 0)),
            pl.BlockSpec((NC, R, DEG_W), lambda i: (0, i, 0)),
            pl.BlockSpec((2 * D, D), lambda i: (0, 0)),
            pl.BlockSpec((1, D), lambda i: (0, 0)),
        ],
        out_specs=pl.BlockSpec((R, D), lambda i: (i, 0)),
        out_shape=jax.ShapeDtypeStruct((N, D), jnp.float32),
    )(h, agg_p, deg_p, w, b)


def kernel(x, edge_index, W1, b1, W2, b2):
    agg1_p, deg_p = _sc_agg_deg(x, edge_index)
    h1 = _tc_layer(True, x, agg1_p, deg_p, W1, b1.reshape(1, D))
    (agg2_p,) = _sc_agg(h1, edge_index)
    out = _tc_layer(False, h1, agg2_p, deg_p, W2, b2.reshape(1, D))
    return out


# speedup vs baseline: 4.9524x; 4.9524x over previous
import functools

import jax
import jax.numpy as jnp
from jax import lax
from jax.experimental import pallas as pl
from jax.experimental.pallas import tpu as pltpu
from jax.experimental.pallas import tpu_sc as plsc

N = 10000
E = 320000
D = 128
DH = D // 2

NC = 2
NS = 16

N_PAD = 10240
RPT = N_PAD // NS
EPT = E // NS
K = 400
SUB = 80
NSUB = K // SUB
NCHUNK = EPT // K
DEG_W = 16

_mesh = plsc.VectorSubcoreMesh(core_axis_name="c", subcore_axis_name="s")
_sc_params = pltpu.CompilerParams(use_tc_tiling_on_sc=False)


def _sc_agg_body(with_deg, *refs):
    if with_deg:
        (x3_hbm, src_hbm, dst_hbm, agg_out, deg_out,
         rows, srcb, dstb, onesb, accum, degacc, sem) = refs
    else:
        (x3_hbm, src_hbm, dst_hbm, agg_out,
         rows, srcb, dstb, accum, sem) = refs

    c = lax.axis_index("c")
    s = lax.axis_index("s")
    rbase = s * RPT

    def zrow(i, carry):
        for j in range(DH // 16):
            rows[i, pl.ds(j * 16, 16)] = jnp.zeros((16,), jnp.float32)
        return carry

    lax.fori_loop(0, K, zrow, 0)
    pltpu.sync_copy(rows, accum.at[pl.ds(rbase, K)])
    pltpu.sync_copy(rows.at[pl.ds(0, RPT - K)],
                    accum.at[pl.ds(rbase + K, RPT - K)])

    if with_deg:
        @pl.when(c == 0)
        def _():
            def zdeg(i, carry):
                onesb[i] = jnp.zeros((16,), jnp.float32)
                return carry

            lax.fori_loop(0, K, zdeg, 0)
            pltpu.sync_copy(onesb, degacc.at[pl.ds(rbase, K)])
            pltpu.sync_copy(onesb.at[pl.ds(0, RPT - K)],
                            degacc.at[pl.ds(rbase + K, RPT - K)])

            def fones(i, carry):
                onesb[i] = jnp.ones((16,), jnp.float32)
                return carry

            lax.fori_loop(0, K, fones, 0)

    plsc.subcore_barrier()

    def chunk(i, carry):
        ebase = s * EPT + i * K
        pltpu.sync_copy(src_hbm.at[pl.ds(ebase, K)], srcb)
        for j in range(NSUB):
            pltpu.sync_copy(dst_hbm.at[pl.ds(ebase + j * SUB, SUB)],
                            dstb.at[j])
        pltpu.async_copy(x3_hbm.at[c].at[srcb], rows, sem).wait()
        for j in range(NSUB):
            pltpu.sync_copy(rows.at[pl.ds(j * SUB, SUB)],
                            accum.at[dstb.at[j]], add=True)
        if with_deg:
            @pl.when(c == 0)
            def _():
                for j in range(NSUB):
                    pltpu.sync_copy(onesb.at[pl.ds(0, SUB)],
                                    degacc.at[dstb.at[j]], add=True)
        return carry

    lax.fori_loop(0, NCHUNK, chunk, 0)

    plsc.subcore_barrier()

    pltpu.sync_copy(accum.at[pl.ds(rbase, RPT)],
                    agg_out.at[c, pl.ds(rbase, RPT)])
    if with_deg:
        @pl.when(c == 0)
        def _():
            pltpu.sync_copy(degacc.at[pl.ds(rbase, RPT)],
                            deg_out.at[pl.ds(rbase, RPT)])


_sc_agg_deg = pl.kernel(
    functools.partial(_sc_agg_body, True),
    out_type=[
        jax.ShapeDtypeStruct((NC, N_PAD, DH), jnp.float32),
        jax.ShapeDtypeStruct((N_PAD, DEG_W), jnp.float32),
    ],
    mesh=_mesh,
    scratch_types=[
        pltpu.VMEM((K, DH), jnp.float32),
        pltpu.VMEM((K,), jnp.int32),
        pltpu.VMEM((NSUB, SUB), jnp.int32),
        pltpu.VMEM((K, DEG_W), jnp.float32),
        pltpu.VMEM_SHARED((N_PAD, DH), jnp.float32),
        pltpu.VMEM_SHARED((N_PAD, DEG_W), jnp.float32),
        pltpu.SemaphoreType.DMA,
    ],
    compiler_params=_sc_params,
)

_sc_agg = pl.kernel(
    functools.partial(_sc_agg_body, False),
    out_type=[jax.ShapeDtypeStruct((NC, N_PAD, DH), jnp.float32)],
    mesh=_mesh,
    scratch_types=[
        pltpu.VMEM((K, DH), jnp.float32),
        pltpu.VMEM((K,), jnp.int32),
        pltpu.VMEM((NSUB, SUB), jnp.int32),
        pltpu.VMEM_SHARED((N_PAD, DH), jnp.float32),
        pltpu.SemaphoreType.DMA,
    ],
    compiler_params=_sc_params,
)

R = 1000


def _tc_layer_body(act, h_ref, p_ref, d_ref, w_ref, b_ref, o_ref):
    p = jnp.concatenate([p_ref[0], p_ref[1]], axis=1)
    deg = jnp.maximum(d_ref[:, 0:1], 1.0)
    agg = p / deg
    z = jnp.concatenate([h_ref[...], agg], axis=1)
    out = jnp.dot(z, w_ref[...], preferred_element_type=jnp.float32)
    out = out + b_ref[...]
    if act:
        out = jnp.maximum(out, 0.0)
    o_ref[...] = out


def _tc_layer(act, h, agg_p, deg, w, b):
    return pl.pallas_call(
        functools.partial(_tc_layer_body, act),
        grid=(N // R,),
        in_specs=[
            pl.BlockSpec((R, D), lambda i: (i, 0)),
            pl.BlockSpec((NC, R, DH), lambda i: (0, i, 0)),
            pl.BlockSpec((R, DEG_W), lambda i: (i, 0)),
            pl.BlockSpec((2 * D, D), lambda i: (0, 0)),
            pl.BlockSpec((1, D), lambda i: (0, 0)),
        ],
        out_specs=pl.BlockSpec((R, D), lambda i: (i, 0)),
        out_shape=jax.ShapeDtypeStruct((N, D), jnp.float32),
    )(h, agg_p, deg, w, b)


def _split_halves(h):
    return jnp.stack([h[:, :DH], h[:, DH:]])


def kernel(x, edge_index, W1, b1, W2, b2):
    src = edge_index[0]
    dst = edge_index[1]
    agg1_p, deg = _sc_agg_deg(_split_halves(x), src, dst)
    h1 = _tc_layer(True, x, agg1_p, deg, W1, b1.reshape(1, D))
    (agg2_p,) = _sc_agg(_split_halves(h1), src, dst)
    out = _tc_layer(False, h1, agg2_p, deg, W2, b2.reshape(1, D))
    return out


# speedup vs baseline: 10.5778x; 2.1359x over previous
import functools

import jax
import jax.numpy as jnp
from jax import lax
from jax.experimental import pallas as pl
from jax.experimental.pallas import tpu as pltpu
from jax.experimental.pallas import tpu_sc as plsc

N = 10000
E = 320000
D = 128
DH = D // 2

NC = 2
NS = 16

N_PAD = 10240
RPT = N_PAD // NS
EPT = E // NS
K = 400
SUB = 100
NSUB = K // SUB
NCHUNK = EPT // K
NITER = NCHUNK // 2
DPT = EPT // SUB
DEG_W = 8

_mesh = plsc.VectorSubcoreMesh(core_axis_name="c", subcore_axis_name="s")
_sc_params = pltpu.CompilerParams(use_tc_tiling_on_sc=False)


def _sc_agg_body(with_deg, *refs):
    if with_deg:
        (x3_hbm, src_hbm, dst2_hbm, agg_out, deg_out,
         rows0, rows1, srcb, dstb, onesb, accum, degacc,
         gsem0, gsem1, ssem0, ssem1, dsem0, dsem1) = refs
    else:
        (x3_hbm, src_hbm, dst2_hbm, agg_out,
         rows0, rows1, srcb, dstb, accum,
         gsem0, gsem1, ssem0, ssem1, dsem0, dsem1) = refs

    c = lax.axis_index("c")
    s = lax.axis_index("s")
    rbase = s * RPT

    def clamp(ch):
        return jnp.where(ch < NCHUNK, ch, 0)

    def start_src(ch, p, sem):
        ch = clamp(ch)
        pltpu.async_copy(src_hbm.at[pl.ds(s * EPT + ch * K, K)],
                         srcb.at[p], sem)

    def wait_src(p, sem):
        pltpu.make_async_copy(src_hbm.at[pl.ds(0, K)], srcb.at[p], sem).wait()

    def start_dst(ch, p, sem):
        ch = clamp(ch)
        pltpu.async_copy(dst2_hbm.at[pl.ds(s * DPT + ch * NSUB, NSUB)],
                         dstb.at[p], sem)

    def wait_dst(p, sem):
        pltpu.make_async_copy(dst2_hbm.at[pl.ds(0, NSUB)],
                              dstb.at[p], sem).wait()

    def start_gather(p, rows, sem):
        pltpu.async_copy(x3_hbm.at[c].at[srcb.at[p]], rows, sem)

    def wait_gather(rows, sem):
        pltpu.make_async_copy(x3_hbm.at[c].at[pl.ds(0, K)], rows, sem).wait()

    def scatter(p, rows, deg_core):
        for j in range(NSUB):
            pltpu.sync_copy(rows.at[pl.ds(j * SUB, SUB)],
                            accum.at[dstb.at[p, j]], add=True)
        if with_deg:
            @pl.when(c == deg_core)
            def _():
                for j in range(NSUB):
                    pltpu.sync_copy(onesb.at[pl.ds(0, SUB)],
                                    degacc.at[dstb.at[p, j]], add=True)

    start_src(0, 0, ssem0)
    start_dst(0, 0, dsem0)
    start_src(1, 1, ssem1)
    start_dst(1, 1, dsem1)

    def zrow(i, carry):
        for j in range(DH // 16):
            rows0[i, pl.ds(j * 16, 16)] = jnp.zeros((16,), jnp.float32)
        return carry

    lax.fori_loop(0, K, zrow, 0)
    pltpu.sync_copy(rows0, accum.at[pl.ds(rbase, K)])
    pltpu.sync_copy(rows0.at[pl.ds(0, RPT - K)],
                    accum.at[pl.ds(rbase + K, RPT - K)])

    if with_deg:
        def zdeg(i, carry):
            onesb[i, pl.ds(0, DEG_W)] = jnp.zeros((DEG_W,), jnp.float32)
            return carry

        lax.fori_loop(0, SUB, zdeg, 0)
        for t in range(RPT // SUB):
            pltpu.sync_copy(onesb, degacc.at[pl.ds(rbase + t * SUB, SUB)])
        rem = RPT % SUB
        if rem:
            pltpu.sync_copy(onesb.at[pl.ds(0, rem)],
                            degacc.at[pl.ds(rbase + RPT - rem, rem)])

        def fones(i, carry):
            onesb[i, pl.ds(0, DEG_W)] = jnp.ones((DEG_W,), jnp.float32)
            return carry

        lax.fori_loop(0, SUB, fones, 0)

    plsc.subcore_barrier()

    wait_src(0, ssem0)
    start_gather(0, rows0, gsem0)

    def body(i, carry):
        e = 2 * i
        wait_src(1, ssem1)
        start_gather(1, rows1, gsem1)
        wait_gather(rows0, gsem0)
        start_src(e + 2, 0, ssem0)
        wait_dst(0, dsem0)
        scatter(0, rows0, 0)
        start_dst(e + 2, 0, dsem0)
        wait_src(0, ssem0)
        start_gather(0, rows0, gsem0)
        wait_gather(rows1, gsem1)
        start_src(e + 3, 1, ssem1)
        wait_dst(1, dsem1)
        scatter(1, rows1, 1)
        start_dst(e + 3, 1, dsem1)
        return carry

    lax.fori_loop(0, NITER, body, 0)

    wait_src(1, ssem1)
    wait_dst(0, dsem0)
    wait_dst(1, dsem1)
    wait_gather(rows0, gsem0)

    plsc.subcore_barrier()

    pltpu.sync_copy(accum.at[pl.ds(rbase, RPT)],
                    agg_out.at[c, pl.ds(rbase, RPT)])
    if with_deg:
        pltpu.sync_copy(degacc.at[pl.ds(rbase, RPT)],
                        deg_out.at[c, pl.ds(rbase, RPT)])


_sc_agg_deg = pl.kernel(
    functools.partial(_sc_agg_body, True),
    out_type=[
        jax.ShapeDtypeStruct((NC, N_PAD, DH), jnp.float32),
        jax.ShapeDtypeStruct((NC, N_PAD, DEG_W), jnp.float32),
    ],
    mesh=_mesh,
    scratch_types=[
        pltpu.VMEM((K, DH), jnp.float32),
        pltpu.VMEM((K, DH), jnp.float32),
        pltpu.VMEM((2, K), jnp.int32),
        pltpu.VMEM((2, NSUB, SUB), jnp.int32),
        pltpu.VMEM((SUB, DEG_W), jnp.float32),
        pltpu.VMEM_SHARED((N_PAD, DH), jnp.float32),
        pltpu.VMEM_SHARED((N_PAD, DEG_W), jnp.float32),
        pltpu.SemaphoreType.DMA,
        pltpu.SemaphoreType.DMA,
        pltpu.SemaphoreType.DMA,
        pltpu.SemaphoreType.DMA,
        pltpu.SemaphoreType.DMA,
        pltpu.SemaphoreType.DMA,
    ],
    compiler_params=_sc_params,
)

_sc_agg = pl.kernel(
    functools.partial(_sc_agg_body, False),
    out_type=[jax.ShapeDtypeStruct((NC, N_PAD, DH), jnp.float32)],
    mesh=_mesh,
    scratch_types=[
        pltpu.VMEM((K, DH), jnp.float32),
        pltpu.VMEM((K, DH), jnp.float32),
        pltpu.VMEM((2, K), jnp.int32),
        pltpu.VMEM((2, NSUB, SUB), jnp.int32),
        pltpu.VMEM_SHARED((N_PAD, DH), jnp.float32),
        pltpu.SemaphoreType.DMA,
        pltpu.SemaphoreType.DMA,
        pltpu.SemaphoreType.DMA,
        pltpu.SemaphoreType.DMA,
        pltpu.SemaphoreType.DMA,
        pltpu.SemaphoreType.DMA,
    ],
    compiler_params=_sc_params,
)

R = 1000


def _tc_layer_body(act, h_ref, p_ref, d_ref, w_ref, b_ref, o_ref):
    p = jnp.concatenate([p_ref[0], p_ref[1]], axis=1)
    deg = jnp.maximum(d_ref[0, :, 0:1] + d_ref[1, :, 0:1], 1.0)
    agg = p / deg
    z = jnp.concatenate([h_ref[...], agg], axis=1)
    out = jnp.dot(z, w_ref[...], preferred_element_type=jnp.float32)
    out = out + b_ref[...]
    if act:
        out = jnp.maximum(out, 0.0)
    o_ref[...] = out


def _tc_layer(act, h, agg_p, deg, w, b):
    return pl.pallas_call(
        functools.partial(_tc_layer_body, act),
        grid=(N // R,),
        in_specs=[
            pl.BlockSpec((R, D), lambda i: (i, 0)),
            pl.BlockSpec((NC, R, DH), lambda i: (0, i, 0)),
            pl.BlockSpec((NC, R, DEG_W), lambda i: (0, i, 0)),
            pl.BlockSpec((2 * D, D), lambda i: (0, 0)),
            pl.BlockSpec((1, D), lambda i: (0, 0)),
        ],
        out_specs=pl.BlockSpec((R, D), lambda i: (i, 0)),
        out_shape=jax.ShapeDtypeStruct((N, D), jnp.float32),
    )(h, agg_p, deg, w, b)


def _split_halves(h):
    return jnp.stack([h[:, :DH], h[:, DH:]])


def kernel(x, edge_index, W1, b1, W2, b2):
    src = edge_index[0]
    dst2 = edge_index[1].reshape(E // SUB, SUB)
    agg1_p, deg = _sc_agg_deg(_split_halves(x), src, dst2)
    h1 = _tc_layer(True, x, agg1_p, deg, W1, b1.reshape(1, D))
    (agg2_p,) = _sc_agg(_split_halves(h1), src, dst2)
    out = _tc_layer(False, h1, agg2_p, deg, W2, b2.reshape(1, D))
    return out


# speedup vs baseline: 11.0204x; 1.0418x over previous
import functools

import jax
import jax.numpy as jnp
from jax import lax
from jax.experimental import pallas as pl
from jax.experimental.pallas import tpu as pltpu
from jax.experimental.pallas import tpu_sc as plsc

N = 10000
E = 320000
D = 128
DH = D // 2

NC = 2
NS = 16

N_PAD = 10240
RPT = N_PAD // NS
EPT = E // NS
K = 400
SUB = 100
NSUB = K // SUB
NCHUNK = EPT // K
NITER = NCHUNK // 2
DPT = EPT // SUB
DEG_W = 8

_mesh = plsc.VectorSubcoreMesh(core_axis_name="c", subcore_axis_name="s")
_sc_params = pltpu.CompilerParams(use_tc_tiling_on_sc=False)


def _sc_agg_body(with_deg, *refs):
    if with_deg:
        (x3_hbm, src_hbm, dst2_hbm, agg_out, deg_out,
         rows0, rows1, srcb, dstb, onesb, accum, degacc,
         gsem0, gsem1, ssem0, ssem1, dsem0, dsem1) = refs
    else:
        (x3_hbm, src_hbm, dst2_hbm, agg_out,
         rows0, rows1, srcb, dstb, accum,
         gsem0, gsem1, ssem0, ssem1, dsem0, dsem1) = refs

    c = lax.axis_index("c")
    s = lax.axis_index("s")
    rbase = s * RPT

    def clamp(ch):
        return jnp.where(ch < NCHUNK, ch, 0)

    def start_src(ch, p, sem):
        ch = clamp(ch)
        pltpu.async_copy(src_hbm.at[pl.ds(s * EPT + ch * K, K)],
                         srcb.at[p], sem)

    def wait_src(p, sem):
        pltpu.make_async_copy(src_hbm.at[pl.ds(0, K)], srcb.at[p], sem).wait()

    def start_dst(ch, p, sem):
        ch = clamp(ch)
        pltpu.async_copy(dst2_hbm.at[pl.ds(s * DPT + ch * NSUB, NSUB)],
                         dstb.at[p], sem)

    def wait_dst(p, sem):
        pltpu.make_async_copy(dst2_hbm.at[pl.ds(0, NSUB)],
                              dstb.at[p], sem).wait()

    def start_gather(p, rows, sem):
        pltpu.async_copy(x3_hbm.at[c].at[srcb.at[p]], rows, sem)

    def wait_gather(rows, sem):
        pltpu.make_async_copy(x3_hbm.at[c].at[pl.ds(0, K)], rows, sem).wait()

    def scatter(p, rows, deg_core):
        for j in range(NSUB):
            pltpu.sync_copy(rows.at[pl.ds(j * SUB, SUB)],
                            accum.at[dstb.at[p, j]], add=True)
        if with_deg:
            @pl.when(c == deg_core)
            def _():
                for j in range(NSUB):
                    pltpu.sync_copy(onesb.at[pl.ds(0, SUB)],
                                    degacc.at[dstb.at[p, j]], add=True)

    start_src(0, 0, ssem0)
    start_dst(0, 0, dsem0)
    start_src(1, 1, ssem1)
    start_dst(1, 1, dsem1)

    def zrow(i, carry):
        for j in range(DH // 16):
            rows0[i, pl.ds(j * 16, 16)] = jnp.zeros((16,), jnp.float32)
        return carry

    lax.fori_loop(0, K, zrow, 0)
    pltpu.sync_copy(rows0, accum.at[pl.ds(rbase, K)])
    pltpu.sync_copy(rows0.at[pl.ds(0, RPT - K)],
                    accum.at[pl.ds(rbase + K, RPT - K)])

    if with_deg:
        def zdeg(i, carry):
            onesb[i, pl.ds(0, DEG_W)] = jnp.zeros((DEG_W,), jnp.float32)
            return carry

        lax.fori_loop(0, SUB, zdeg, 0)
        for t in range(RPT // SUB):
            pltpu.sync_copy(onesb, degacc.at[pl.ds(rbase + t * SUB, SUB)])
        rem = RPT % SUB
        if rem:
            pltpu.sync_copy(onesb.at[pl.ds(0, rem)],
                            degacc.at[pl.ds(rbase + RPT - rem, rem)])

        def fones(i, carry):
            onesb[i, pl.ds(0, DEG_W)] = jnp.ones((DEG_W,), jnp.float32)
            return carry

        lax.fori_loop(0, SUB, fones, 0)

    plsc.subcore_barrier()

    wait_src(0, ssem0)
    start_gather(0, rows0, gsem0)

    def body(i, carry):
        e = 2 * i
        wait_src(1, ssem1)
        start_gather(1, rows1, gsem1)
        wait_gather(rows0, gsem0)
        start_src(e + 2, 0, ssem0)
        wait_dst(0, dsem0)
        scatter(0, rows0, 0)
        start_dst(e + 2, 0, dsem0)
        wait_src(0, ssem0)
        start_gather(0, rows0, gsem0)
        wait_gather(rows1, gsem1)
        start_src(e + 3, 1, ssem1)
        wait_dst(1, dsem1)
        scatter(1, rows1, 1)
        start_dst(e + 3, 1, dsem1)
        return carry

    lax.fori_loop(0, NITER, body, 0)

    wait_src(1, ssem1)
    wait_dst(0, dsem0)
    wait_dst(1, dsem1)
    wait_gather(rows0, gsem0)

    plsc.subcore_barrier()

    pltpu.sync_copy(accum.at[pl.ds(rbase, RPT)],
                    agg_out.at[c, pl.ds(rbase, RPT)])
    if with_deg:
        pltpu.sync_copy(degacc.at[pl.ds(rbase, RPT)],
                        deg_out.at[c, pl.ds(rbase, RPT)])


_sc_agg_deg = pl.kernel(
    functools.partial(_sc_agg_body, True),
    out_type=[
        jax.ShapeDtypeStruct((NC, N_PAD, DH), jnp.float32),
        jax.ShapeDtypeStruct((NC, N_PAD, DEG_W), jnp.float32),
    ],
    mesh=_mesh,
    scratch_types=[
        pltpu.VMEM((K, DH), jnp.float32),
        pltpu.VMEM((K, DH), jnp.float32),
        pltpu.VMEM((2, K), jnp.int32),
        pltpu.VMEM((2, NSUB, SUB), jnp.int32),
        pltpu.VMEM((SUB, DEG_W), jnp.float32),
        pltpu.VMEM_SHARED((N_PAD, DH), jnp.float32),
        pltpu.VMEM_SHARED((N_PAD, DEG_W), jnp.float32),
        pltpu.SemaphoreType.DMA,
        pltpu.SemaphoreType.DMA,
        pltpu.SemaphoreType.DMA,
        pltpu.SemaphoreType.DMA,
        pltpu.SemaphoreType.DMA,
        pltpu.SemaphoreType.DMA,
    ],
    compiler_params=_sc_params,
)

_sc_agg = pl.kernel(
    functools.partial(_sc_agg_body, False),
    out_type=[jax.ShapeDtypeStruct((NC, N_PAD, DH), jnp.float32)],
    mesh=_mesh,
    scratch_types=[
        pltpu.VMEM((K, DH), jnp.float32),
        pltpu.VMEM((K, DH), jnp.float32),
        pltpu.VMEM((2, K), jnp.int32),
        pltpu.VMEM((2, NSUB, SUB), jnp.int32),
        pltpu.VMEM_SHARED((N_PAD, DH), jnp.float32),
        pltpu.SemaphoreType.DMA,
        pltpu.SemaphoreType.DMA,
        pltpu.SemaphoreType.DMA,
        pltpu.SemaphoreType.DMA,
        pltpu.SemaphoreType.DMA,
        pltpu.SemaphoreType.DMA,
    ],
    compiler_params=_sc_params,
)

R = 1000


def _tc_self_body(h_ref, w_ref, b_ref, o_ref):
    h = h_ref[...]
    o_ref[...] = (
        jnp.dot(h, w_ref[...], preferred_element_type=jnp.float32)
        + b_ref[...])


def _tc_self(h, w, b):
    return pl.pallas_call(
        _tc_self_body,
        grid=(N // R,),
        in_specs=[
            pl.BlockSpec((R, D), lambda i: (i, 0)),
            pl.BlockSpec((D, D), lambda i: (0, 0)),
            pl.BlockSpec((1, D), lambda i: (0, 0)),
        ],
        out_specs=pl.BlockSpec((R, D), lambda i: (i, 0)),
        out_shape=jax.ShapeDtypeStruct((N, D), jnp.float32),
    )(h, w, b)


def _tc_self_split_body(h_ref, w_ref, b_ref, o_ref):
    h = jnp.concatenate([h_ref[0], h_ref[1]], axis=1)
    o_ref[...] = (
        jnp.dot(h, w_ref[...], preferred_element_type=jnp.float32)
        + b_ref[...])


def _tc_self_split(h3, w, b):
    return pl.pallas_call(
        _tc_self_split_body,
        grid=(N // R,),
        in_specs=[
            pl.BlockSpec((NC, R, DH), lambda i: (0, i, 0)),
            pl.BlockSpec((D, D), lambda i: (0, 0)),
            pl.BlockSpec((1, D), lambda i: (0, 0)),
        ],
        out_specs=pl.BlockSpec((R, D), lambda i: (i, 0)),
        out_shape=jax.ShapeDtypeStruct((N, D), jnp.float32),
    )(h3, w, b)


def _tc_combine_body(act, split_out, pre_ref, p_ref, d_ref, w_ref, o_ref):
    agg = jnp.concatenate([p_ref[0], p_ref[1]], axis=1)
    deg = jnp.maximum(d_ref[0, :, 0:1] + d_ref[1, :, 0:1], 1.0)
    agg = agg / deg
    out = pre_ref[...] + jnp.dot(agg, w_ref[...],
                                 preferred_element_type=jnp.float32)
    if act:
        out = jnp.maximum(out, 0.0)
    if split_out:
        o_ref[0] = out[:, :DH]
        o_ref[1] = out[:, DH:]
    else:
        o_ref[...] = out


def _tc_combine(act, split_out, pre, agg_p, deg, w_agg):
    if split_out:
        out_shape = jax.ShapeDtypeStruct((NC, N, DH), jnp.float32)
        out_spec = pl.BlockSpec((NC, R, DH), lambda i: (0, i, 0))
    else:
        out_shape = jax.ShapeDtypeStruct((N, D), jnp.float32)
        out_spec = pl.BlockSpec((R, D), lambda i: (i, 0))
    return pl.pallas_call(
        functools.partial(_tc_combine_body, act, split_out),
        grid=(N // R,),
        in_specs=[
            pl.BlockSpec((R, D), lambda i: (i, 0)),
            pl.BlockSpec((NC, R, DH), lambda i: (0, i, 0)),
            pl.BlockSpec((NC, R, DEG_W), lambda i: (0, i, 0)),
            pl.BlockSpec((D, D), lambda i: (0, 0)),
        ],
        out_specs=out_spec,
        out_shape=out_shape,
    )(pre, agg_p, deg, w_agg)


def kernel(x, edge_index, W1, b1, W2, b2):
    src = edge_index[0]
    dst2 = edge_index[1].reshape(E // SUB, SUB)
    x3 = jnp.stack([x[:, :DH], x[:, DH:]])
    agg1_p, deg = _sc_agg_deg(x3, src, dst2)
    pre1 = _tc_self(x, W1[:D], b1.reshape(1, D))
    h1_3 = _tc_combine(True, True, pre1, agg1_p, deg, W1[D:])
    (agg2_p,) = _sc_agg(h1_3, src, dst2)
    pre2 = _tc_self_split(h1_3, W2[:D], b2.reshape(1, D))
    out = _tc_combine(False, False, pre2, agg2_p, deg, W2[D:])
    return out


# speedup vs baseline: 11.1298x; 1.0099x over previous
import functools

import jax
import jax.numpy as jnp
from jax import lax
from jax.experimental import pallas as pl
from jax.experimental.pallas import tpu as pltpu
from jax.experimental.pallas import tpu_sc as plsc

N = 10000
E = 320000
D = 128
DH = D // 2

NC = 2
NS = 16

N_PAD = 10240
RPT = N_PAD // NS
EPT = E // NS
K = 400
SUB = 100
NSUB = K // SUB
NCHUNK = EPT // K
NITER = NCHUNK // 2
DPT = EPT // SUB
DEG_W = 8

_mesh = plsc.VectorSubcoreMesh(core_axis_name="c", subcore_axis_name="s")
_sc_params = pltpu.CompilerParams(use_tc_tiling_on_sc=False)


def _sc_agg_body(with_deg, *refs):
    if with_deg:
        (x3_hbm, src_hbm, dst2_hbm, agg_out, deg_out,
         rows0, rows1, srcb, dstb, onesb, accum, degacc,
         gsem0, gsem1, ssem0, ssem1, dsem0, dsem1) = refs
    else:
        (x3_hbm, src_hbm, dst2_hbm, agg_out,
         rows0, rows1, srcb, dstb, accum,
         gsem0, gsem1, ssem0, ssem1, dsem0, dsem1) = refs

    c = lax.axis_index("c")
    s = lax.axis_index("s")
    rbase = s * RPT

    def clamp(ch):
        return jnp.where(ch < NCHUNK, ch, 0)

    def start_src(ch, p, sem):
        ch = clamp(ch)
        pltpu.async_copy(src_hbm.at[pl.ds(s * EPT + ch * K, K)],
                         srcb.at[p], sem)

    def wait_src(p, sem):
        pltpu.make_async_copy(src_hbm.at[pl.ds(0, K)], srcb.at[p], sem).wait()

    def start_dst(ch, p, sem):
        ch = clamp(ch)
        pltpu.async_copy(dst2_hbm.at[pl.ds(s * DPT + ch * NSUB, NSUB)],
                         dstb.at[p], sem)

    def wait_dst(p, sem):
        pltpu.make_async_copy(dst2_hbm.at[pl.ds(0, NSUB)],
                              dstb.at[p], sem).wait()

    def start_gather(p, rows, sem):
        pltpu.async_copy(x3_hbm.at[c].at[srcb.at[p]], rows, sem)

    def wait_gather(rows, sem):
        pltpu.make_async_copy(x3_hbm.at[c].at[pl.ds(0, K)], rows, sem).wait()

    def scatter(p, rows, deg_core):
        for j in range(NSUB):
            pltpu.sync_copy(rows.at[pl.ds(j * SUB, SUB)],
                            accum.at[dstb.at[p, j]], add=True)
        if with_deg:
            @pl.when(c == deg_core)
            def _():
                for j in range(NSUB):
                    pltpu.sync_copy(onesb.at[pl.ds(0, SUB)],
                                    degacc.at[dstb.at[p, j]], add=True)

    start_src(0, 0, ssem0)
    start_dst(0, 0, dsem0)
    start_src(1, 1, ssem1)
    start_dst(1, 1, dsem1)

    def zrow(i, carry):
        for j in range(DH // 16):
            rows0[i, pl.ds(j * 16, 16)] = jnp.zeros((16,), jnp.float32)
        return carry

    lax.fori_loop(0, K, zrow, 0)
    pltpu.sync_copy(rows0, accum.at[pl.ds(rbase, K)])
    pltpu.sync_copy(rows0.at[pl.ds(0, RPT - K)],
                    accum.at[pl.ds(rbase + K, RPT - K)])

    if with_deg:
        def zdeg(i, carry):
            onesb[i, pl.ds(0, DEG_W)] = jnp.zeros((DEG_W,), jnp.float32)
            return carry

        lax.fori_loop(0, SUB, zdeg, 0)
        for t in range(RPT // SUB):
            pltpu.sync_copy(onesb, degacc.at[pl.ds(rbase + t * SUB, SUB)])
        rem = RPT % SUB
        if rem:
            pltpu.sync_copy(onesb.at[pl.ds(0, rem)],
                            degacc.at[pl.ds(rbase + RPT - rem, rem)])

        def fones(i, carry):
            onesb[i, pl.ds(0, DEG_W)] = jnp.ones((DEG_W,), jnp.float32)
            return carry

        lax.fori_loop(0, SUB, fones, 0)

    plsc.subcore_barrier()

    wait_src(0, ssem0)
    start_gather(0, rows0, gsem0)

    def body(i, carry):
        e = 2 * i
        wait_src(1, ssem1)
        start_gather(1, rows1, gsem1)
        wait_gather(rows0, gsem0)
        start_src(e + 2, 0, ssem0)
        wait_dst(0, dsem0)
        scatter(0, rows0, 0)
        start_dst(e + 2, 0, dsem0)
        wait_src(0, ssem0)
        start_gather(0, rows0, gsem0)
        wait_gather(rows1, gsem1)
        start_src(e + 3, 1, ssem1)
        wait_dst(1, dsem1)
        scatter(1, rows1, 1)
        start_dst(e + 3, 1, dsem1)
        return carry

    lax.fori_loop(0, NITER, body, 0)

    wait_src(1, ssem1)
    wait_dst(0, dsem0)
    wait_dst(1, dsem1)
    wait_gather(rows0, gsem0)

    plsc.subcore_barrier()

    pltpu.sync_copy(accum.at[pl.ds(rbase, RPT)],
                    agg_out.at[c, pl.ds(rbase, RPT)])
    if with_deg:
        pltpu.sync_copy(degacc.at[pl.ds(rbase, RPT)],
                        deg_out.at[c, pl.ds(rbase, RPT)])


_sc_agg_deg = pl.kernel(
    functools.partial(_sc_agg_body, True),
    out_type=[
        jax.ShapeDtypeStruct((NC, N_PAD, DH), jnp.float32),
        jax.ShapeDtypeStruct((NC, N_PAD, DEG_W), jnp.float32),
    ],
    mesh=_mesh,
    scratch_types=[
        pltpu.VMEM((K, DH), jnp.float32),
        pltpu.VMEM((K, DH), jnp.float32),
        pltpu.VMEM((2, K), jnp.int32),
        pltpu.VMEM((2, NSUB, SUB), jnp.int32),
        pltpu.VMEM((SUB, DEG_W), jnp.float32),
        pltpu.VMEM_SHARED((N_PAD, DH), jnp.float32),
        pltpu.VMEM_SHARED((N_PAD, DEG_W), jnp.float32),
        pltpu.SemaphoreType.DMA,
        pltpu.SemaphoreType.DMA,
        pltpu.SemaphoreType.DMA,
        pltpu.SemaphoreType.DMA,
        pltpu.SemaphoreType.DMA,
        pltpu.SemaphoreType.DMA,
    ],
    compiler_params=_sc_params,
)

_sc_agg = pl.kernel(
    functools.partial(_sc_agg_body, False),
    out_type=[jax.ShapeDtypeStruct((NC, N_PAD, DH), jnp.float32)],
    mesh=_mesh,
    scratch_types=[
        pltpu.VMEM((K, DH), jnp.float32),
        pltpu.VMEM((K, DH), jnp.float32),
        pltpu.VMEM((2, K), jnp.int32),
        pltpu.VMEM((2, NSUB, SUB), jnp.int32),
        pltpu.VMEM_SHARED((N_PAD, DH), jnp.float32),
        pltpu.SemaphoreType.DMA,
        pltpu.SemaphoreType.DMA,
        pltpu.SemaphoreType.DMA,
        pltpu.SemaphoreType.DMA,
        pltpu.SemaphoreType.DMA,
        pltpu.SemaphoreType.DMA,
    ],
    compiler_params=_sc_params,
)

R = 2000


def _tc_self_body(h_ref, w_ref, b_ref, o_ref):
    h = h_ref[...]
    o_ref[...] = (
        jnp.dot(h, w_ref[...], preferred_element_type=jnp.float32)
        + b_ref[...])


def _tc_self(h, w, b):
    return pl.pallas_call(
        _tc_self_body,
        grid=(N // R,),
        in_specs=[
            pl.BlockSpec((R, D), lambda i: (i, 0)),
            pl.BlockSpec((D, D), lambda i: (0, 0)),
            pl.BlockSpec((1, D), lambda i: (0, 0)),
        ],
        out_specs=pl.BlockSpec((R, D), lambda i: (i, 0)),
        out_shape=jax.ShapeDtypeStruct((N, D), jnp.float32),
    )(h, w, b)


def _tc_self_split_body(h_ref, w_ref, b_ref, o_ref):
    h = jnp.concatenate([h_ref[0], h_ref[1]], axis=1)
    o_ref[...] = (
        jnp.dot(h, w_ref[...], preferred_element_type=jnp.float32)
        + b_ref[...])


def _tc_self_split(h3, w, b):
    return pl.pallas_call(
        _tc_self_split_body,
        grid=(N // R,),
        in_specs=[
            pl.BlockSpec((NC, R, DH), lambda i: (0, i, 0)),
            pl.BlockSpec((D, D), lambda i: (0, 0)),
            pl.BlockSpec((1, D), lambda i: (0, 0)),
        ],
        out_specs=pl.BlockSpec((R, D), lambda i: (i, 0)),
        out_shape=jax.ShapeDtypeStruct((N, D), jnp.float32),
    )(h3, w, b)


def _tc_combine_body(act, split_out, pre_ref, p_ref, d_ref, w_ref, o_ref):
    agg = jnp.concatenate([p_ref[0], p_ref[1]], axis=1)
    deg = jnp.maximum(d_ref[0, :, 0:1] + d_ref[1, :, 0:1], 1.0)
    agg = agg / deg
    out = pre_ref[...] + jnp.dot(agg, w_ref[...],
                                 preferred_element_type=jnp.float32)
    if act:
        out = jnp.maximum(out, 0.0)
    if split_out:
        o_ref[0] = out[:, :DH]
        o_ref[1] = out[:, DH:]
    else:
        o_ref[...] = out


def _tc_combine(act, split_out, pre, agg_p, deg, w_agg):
    if split_out:
        out_shape = jax.ShapeDtypeStruct((NC, N, DH), jnp.float32)
        out_spec = pl.BlockSpec((NC, R, DH), lambda i: (0, i, 0))
    else:
        out_shape = jax.ShapeDtypeStruct((N, D), jnp.float32)
        out_spec = pl.BlockSpec((R, D), lambda i: (i, 0))
    return pl.pallas_call(
        functools.partial(_tc_combine_body, act, split_out),
        grid=(N // R,),
        in_specs=[
            pl.BlockSpec((R, D), lambda i: (i, 0)),
            pl.BlockSpec((NC, R, DH), lambda i: (0, i, 0)),
            pl.BlockSpec((NC, R, DEG_W), lambda i: (0, i, 0)),
            pl.BlockSpec((D, D), lambda i: (0, 0)),
        ],
        out_specs=out_spec,
        out_shape=out_shape,
    )(pre, agg_p, deg, w_agg)


def kernel(x, edge_index, W1, b1, W2, b2):
    src = edge_index[0]
    dst2 = edge_index[1].reshape(E // SUB, SUB)
    x3 = jnp.stack([x[:, :DH], x[:, DH:]])
    agg1_p, deg = _sc_agg_deg(x3, src, dst2)
    pre1 = _tc_self(x, W1[:D], b1.reshape(1, D))
    h1_3 = _tc_combine(True, True, pre1, agg1_p, deg, W1[D:])
    (agg2_p,) = _sc_agg(h1_3, src, dst2)
    pre2 = _tc_self_split(h1_3, W2[:D], b2.reshape(1, D))
    out = _tc_combine(False, False, pre2, agg2_p, deg, W2[D:])
    return out


# speedup vs baseline: 11.2302x; 1.0090x over previous
import functools

import jax
import jax.numpy as jnp
from jax import lax
from jax.experimental import pallas as pl
from jax.experimental.pallas import tpu as pltpu
from jax.experimental.pallas import tpu_sc as plsc

N = 10000
E = 320000
D = 128
DH = D // 2

NC = 2
NS = 16

N_PAD = 10240
RPT = N_PAD // NS
EPT = E // NS
K = 400
SUB = 80
NSUB = K // SUB
NCHUNK = EPT // K
NITER = NCHUNK // 2
DEG_W = 8

_mesh = plsc.VectorSubcoreMesh(core_axis_name="c", subcore_axis_name="s")
_sc_params = pltpu.CompilerParams(use_tc_tiling_on_sc=False)


def _sc_agg_body(with_deg, *refs):
    if with_deg:
        (x3_hbm, src_hbm, dst_hbm, agg_out, deg_out,
         rows0, rows1, srcb, dstb, onesb, accum, degacc,
         gsem0, gsem1, ssem0, ssem1, dsem0, dsem1) = refs
    else:
        (x3_hbm, src_hbm, dst_hbm, agg_out,
         rows0, rows1, srcb, dstb, accum,
         gsem0, gsem1, ssem0, ssem1, dsem0, dsem1) = refs

    c = lax.axis_index("c")
    s = lax.axis_index("s")
    rbase = s * RPT

    def clamp(ch):
        return jnp.where(ch < NCHUNK, ch, 0)

    def start_src(ch, p, sem):
        ch = clamp(ch)
        pltpu.async_copy(src_hbm.at[pl.ds(s * EPT + ch * K, K)],
                         srcb.at[p], sem)

    def wait_src(p, sem):
        pltpu.make_async_copy(src_hbm.at[pl.ds(0, K)], srcb.at[p], sem).wait()

    def start_dst(ch, p, sem):
        ch = clamp(ch)
        for j in range(NSUB):
            pltpu.async_copy(
                dst_hbm.at[pl.ds(s * EPT + ch * K + j * SUB, SUB)],
                dstb.at[p, j], sem)

    def wait_dst(p, sem):
        for j in range(NSUB):
            pltpu.make_async_copy(dst_hbm.at[pl.ds(0, SUB)],
                                  dstb.at[p, j], sem).wait()

    def start_gather(p, rows, sem):
        pltpu.async_copy(x3_hbm.at[c].at[srcb.at[p]], rows, sem)

    def wait_gather(rows, sem):
        pltpu.make_async_copy(x3_hbm.at[c].at[pl.ds(0, K)], rows, sem).wait()

    def scatter(p, rows, deg_core):
        for j in range(NSUB):
            pltpu.sync_copy(rows.at[pl.ds(j * SUB, SUB)],
                            accum.at[dstb.at[p, j]], add=True)
        if with_deg:
            @pl.when(c == deg_core)
            def _():
                for j in range(NSUB):
                    pltpu.sync_copy(onesb.at[pl.ds(0, SUB)],
                                    degacc.at[dstb.at[p, j]], add=True)

    start_src(0, 0, ssem0)
    start_dst(0, 0, dsem0)
    start_src(1, 1, ssem1)
    start_dst(1, 1, dsem1)

    def zrow(i, carry):
        for j in range(DH // 16):
            rows0[i, pl.ds(j * 16, 16)] = jnp.zeros((16,), jnp.float32)
        return carry

    lax.fori_loop(0, K, zrow, 0)
    pltpu.sync_copy(rows0, accum.at[pl.ds(rbase, K)])
    pltpu.sync_copy(rows0.at[pl.ds(0, RPT - K)],
                    accum.at[pl.ds(rbase + K, RPT - K)])

    if with_deg:
        def zdeg(i, carry):
            onesb[i, pl.ds(0, DEG_W)] = jnp.zeros((DEG_W,), jnp.float32)
            return carry

        lax.fori_loop(0, SUB, zdeg, 0)
        for t in range(RPT // SUB):
            pltpu.sync_copy(onesb, degacc.at[pl.ds(rbase + t * SUB, SUB)])
        rem = RPT % SUB
        if rem:
            pltpu.sync_copy(onesb.at[pl.ds(0, rem)],
                            degacc.at[pl.ds(rbase + RPT - rem, rem)])

        def fones(i, carry):
            onesb[i, pl.ds(0, DEG_W)] = jnp.ones((DEG_W,), jnp.float32)
            return carry

        lax.fori_loop(0, SUB, fones, 0)

    plsc.subcore_barrier()

    wait_src(0, ssem0)
    start_gather(0, rows0, gsem0)

    def body(i, carry):
        e = 2 * i
        wait_src(1, ssem1)
        start_gather(1, rows1, gsem1)
        wait_gather(rows0, gsem0)
        start_src(e + 2, 0, ssem0)
        wait_dst(0, dsem0)
        scatter(0, rows0, 0)
        start_dst(e + 2, 0, dsem0)
        wait_src(0, ssem0)
        start_gather(0, rows0, gsem0)
        wait_gather(rows1, gsem1)
        start_src(e + 3, 1, ssem1)
        wait_dst(1, dsem1)
        scatter(1, rows1, 1)
        start_dst(e + 3, 1, dsem1)
        return carry

    lax.fori_loop(0, NITER, body, 0)

    wait_src(1, ssem1)
    wait_dst(0, dsem0)
    wait_dst(1, dsem1)
    wait_gather(rows0, gsem0)

    plsc.subcore_barrier()

    pltpu.sync_copy(accum.at[pl.ds(rbase, RPT)],
                    agg_out.at[c, pl.ds(rbase, RPT)])
    if with_deg:
        pltpu.sync_copy(degacc.at[pl.ds(rbase, RPT)],
                        deg_out.at[c, pl.ds(rbase, RPT)])


_sc_agg_deg = pl.kernel(
    functools.partial(_sc_agg_body, True),
    out_type=[
        jax.ShapeDtypeStruct((NC, N_PAD, DH), jnp.float32),
        jax.ShapeDtypeStruct((NC, N_PAD, DEG_W), jnp.float32),
    ],
    mesh=_mesh,
    scratch_types=[
        pltpu.VMEM((K, DH), jnp.float32),
        pltpu.VMEM((K, DH), jnp.float32),
        pltpu.VMEM((2, K), jnp.int32),
        pltpu.VMEM((2, NSUB, SUB), jnp.int32),
        pltpu.VMEM((SUB, DEG_W), jnp.float32),
        pltpu.VMEM_SHARED((N_PAD, DH), jnp.float32),
        pltpu.VMEM_SHARED((N_PAD, DEG_W), jnp.float32),
        pltpu.SemaphoreType.DMA,
        pltpu.SemaphoreType.DMA,
        pltpu.SemaphoreType.DMA,
        pltpu.SemaphoreType.DMA,
        pltpu.SemaphoreType.DMA,
        pltpu.SemaphoreType.DMA,
    ],
    compiler_params=_sc_params,
)

_sc_agg = pl.kernel(
    functools.partial(_sc_agg_body, False),
    out_type=[jax.ShapeDtypeStruct((NC, N_PAD, DH), jnp.float32)],
    mesh=_mesh,
    scratch_types=[
        pltpu.VMEM((K, DH), jnp.float32),
        pltpu.VMEM((K, DH), jnp.float32),
        pltpu.VMEM((2, K), jnp.int32),
        pltpu.VMEM((2, NSUB, SUB), jnp.int32),
        pltpu.VMEM_SHARED((N_PAD, DH), jnp.float32),
        pltpu.SemaphoreType.DMA,
        pltpu.SemaphoreType.DMA,
        pltpu.SemaphoreType.DMA,
        pltpu.SemaphoreType.DMA,
        pltpu.SemaphoreType.DMA,
        pltpu.SemaphoreType.DMA,
    ],
    compiler_params=_sc_params,
)

R = 2000


def _tc_split_body(h_ref, o_ref):
    h = h_ref[...]
    o_ref[0] = h[:, :DH]
    o_ref[1] = h[:, DH:]


def _tc_split(h):
    return pl.pallas_call(
        _tc_split_body,
        grid=(N // R,),
        in_specs=[pl.BlockSpec((R, D), lambda i: (i, 0))],
        out_specs=pl.BlockSpec((NC, R, DH), lambda i: (0, i, 0)),
        out_shape=jax.ShapeDtypeStruct((NC, N, DH), jnp.float32),
    )(h)


def _tc_self_body(h_ref, w_ref, b_ref, o_ref):
    h = h_ref[...]
    o_ref[...] = (
        jnp.dot(h, w_ref[...], preferred_element_type=jnp.float32)
        + b_ref[...])


def _tc_self(h, w, b):
    return pl.pallas_call(
        _tc_self_body,
        grid=(N // R,),
        in_specs=[
            pl.BlockSpec((R, D), lambda i: (i, 0)),
            pl.BlockSpec((D, D), lambda i: (0, 0)),
            pl.BlockSpec((1, D), lambda i: (0, 0)),
        ],
        out_specs=pl.BlockSpec((R, D), lambda i: (i, 0)),
        out_shape=jax.ShapeDtypeStruct((N, D), jnp.float32),
    )(h, w, b)


def _tc_self_split_body(h_ref, w_ref, b_ref, o_ref):
    h = jnp.concatenate([h_ref[0], h_ref[1]], axis=1)
    o_ref[...] = (
        jnp.dot(h, w_ref[...], preferred_element_type=jnp.float32)
        + b_ref[...])


def _tc_self_split(h3, w, b):
    return pl.pallas_call(
        _tc_self_split_body,
        grid=(N // R,),
        in_specs=[
            pl.BlockSpec((NC, R, DH), lambda i: (0, i, 0)),
            pl.BlockSpec((D, D), lambda i: (0, 0)),
            pl.BlockSpec((1, D), lambda i: (0, 0)),
        ],
        out_specs=pl.BlockSpec((R, D), lambda i: (i, 0)),
        out_shape=jax.ShapeDtypeStruct((N, D), jnp.float32),
    )(h3, w, b)


def _tc_combine_body(act, split_out, pre_ref, p_ref, d_ref, w_ref, o_ref):
    agg = jnp.concatenate([p_ref[0], p_ref[1]], axis=1)
    deg = jnp.maximum(d_ref[0, :, 0:1] + d_ref[1, :, 0:1], 1.0)
    agg = agg / deg
    out = pre_ref[...] + jnp.dot(agg, w_ref[...],
                                 preferred_element_type=jnp.float32)
    if act:
        out = jnp.maximum(out, 0.0)
    if split_out:
        o_ref[0] = out[:, :DH]
        o_ref[1] = out[:, DH:]
    else:
        o_ref[...] = out


def _tc_combine(act, split_out, pre, agg_p, deg, w_agg):
    if split_out:
        out_shape = jax.ShapeDtypeStruct((NC, N, DH), jnp.float32)
        out_spec = pl.BlockSpec((NC, R, DH), lambda i: (0, i, 0))
    else:
        out_shape = jax.ShapeDtypeStruct((N, D), jnp.float32)
        out_spec = pl.BlockSpec((R, D), lambda i: (i, 0))
    return pl.pallas_call(
        functools.partial(_tc_combine_body, act, split_out),
        grid=(N // R,),
        in_specs=[
            pl.BlockSpec((R, D), lambda i: (i, 0)),
            pl.BlockSpec((NC, R, DH), lambda i: (0, i, 0)),
            pl.BlockSpec((NC, R, DEG_W), lambda i: (0, i, 0)),
            pl.BlockSpec((D, D), lambda i: (0, 0)),
        ],
        out_specs=out_spec,
        out_shape=out_shape,
    )(pre, agg_p, deg, w_agg)


def kernel(x, edge_index, W1, b1, W2, b2):
    src = edge_index[0]
    dst = edge_index[1]
    x3 = _tc_split(x)
    agg1_p, deg = _sc_agg_deg(x3, src, dst)
    pre1 = _tc_self(x, W1[:D], b1.reshape(1, D))
    h1_3 = _tc_combine(True, True, pre1, agg1_p, deg, W1[D:])
    (agg2_p,) = _sc_agg(h1_3, src, dst)
    pre2 = _tc_self_split(h1_3, W2[:D], b2.reshape(1, D))
    out = _tc_combine(False, False, pre2, agg2_p, deg, W2[D:])
    return out
